# baseline (speedup 1.0000x reference)
import jax
import jax.numpy as jnp
from jax.experimental import pallas as pl
from jax.experimental.pallas import tpu as pltpu

_BM = 1024
_NBUF = 4


def _gate_kernel(x_hbm, w_ref, o_ref, xbuf, sems):
    T, D = x_hbm.shape
    nblk = T // _BM

    def _copy(blk, slot):
        return pltpu.make_async_copy(
            x_hbm.at[pl.ds(blk * _BM, _BM), :], xbuf.at[slot], sems.at[slot])

    for s in range(min(_NBUF, nblk)):
        _copy(s, s).start()
    for i in range(nblk):
        slot = i % _NBUF
        _copy(i, slot).wait()
        o_ref[pl.ds(i * _BM, _BM), :] = jax.lax.dot_general(
            xbuf[slot], w_ref[:],
            dimension_numbers=(((1,), (1,)), ((), ())),
            preferred_element_type=jnp.float32,
        )
        nxt = i + _NBUF
        if nxt < nblk:
            _copy(nxt, slot).start()


def kernel(x, W):
    T, D = x.shape
    E = W.shape[0]
    return pl.pallas_call(
        _gate_kernel,
        in_specs=[
            pl.BlockSpec(memory_space=pltpu.MemorySpace.HBM),
            pl.BlockSpec((E, D), lambda: (0, 0)),
        ],
        out_specs=pl.BlockSpec((T, E), lambda: (0, 0)),
        out_shape=jax.ShapeDtypeStruct((T, E), jnp.float32),
        scratch_shapes=[
            pltpu.VMEM((_NBUF, _BM, D), jnp.float32),
            pltpu.SemaphoreType.DMA((_NBUF,)),
        ],
    )(x, W)


# speedup vs baseline: 1.0474x; 1.0474x over previous
import jax
import jax.numpy as jnp
from jax.experimental import pallas as pl
from jax.experimental.pallas import tpu as pltpu

_BM = 1024
_NBUF = 4


def _body(x_ref, w_ref, o_ref):
    o_ref[:] = jax.lax.dot_general(
        x_ref[:], w_ref[:],
        dimension_numbers=(((1,), (1,)), ((), ())),
        preferred_element_type=jnp.float32,
    )


def _outer(x_hbm, w_hbm, o_hbm):
    T, D = x_hbm.shape
    E = w_hbm.shape[0]
    pipe = pltpu.emit_pipeline(
        _body,
        grid=(T // _BM,),
        in_specs=[
            pl.BlockSpec((_BM, D), lambda i: (i, 0),
                         pipeline_mode=pl.Buffered(buffer_count=_NBUF)),
            pl.BlockSpec((E, D), lambda i: (0, 0)),
        ],
        out_specs=[pl.BlockSpec((_BM, E), lambda i: (i, 0))],
    )
    pipe(x_hbm, w_hbm, o_hbm)


def kernel(x, W):
    T, D = x.shape
    E = W.shape[0]
    return pl.pallas_call(
        _outer,
        in_specs=[
            pl.BlockSpec(memory_space=pltpu.MemorySpace.HBM),
            pl.BlockSpec(memory_space=pltpu.MemorySpace.HBM),
        ],
        out_specs=pl.BlockSpec(memory_space=pltpu.MemorySpace.HBM),
        out_shape=jax.ShapeDtypeStruct((T, E), jnp.float32),
    )(x, W)


# speedup vs baseline: 1.0662x; 1.0179x over previous
import jax
import jax.numpy as jnp
from jax.experimental import pallas as pl
from jax.experimental.pallas import tpu as pltpu

_BM = 512
_NBUF = 8


def _body(x_ref, w_ref, o_ref):
    o_ref[:] = jax.lax.dot_general(
        x_ref[:], w_ref[:],
        dimension_numbers=(((1,), (1,)), ((), ())),
        preferred_element_type=jnp.float32,
    )


def _outer(x_hbm, w_hbm, o_hbm):
    T, D = x_hbm.shape
    E = w_hbm.shape[0]
    pipe = pltpu.emit_pipeline(
        _body,
        grid=(T // _BM,),
        in_specs=[
            pl.BlockSpec((_BM, D), lambda i: (i, 0),
                         pipeline_mode=pl.Buffered(buffer_count=_NBUF)),
            pl.BlockSpec((E, D), lambda i: (0, 0)),
        ],
        out_specs=[pl.BlockSpec((_BM, E), lambda i: (i, 0))],
    )
    pipe(x_hbm, w_hbm, o_hbm)


def kernel(x, W):
    T, D = x.shape
    E = W.shape[0]
    return pl.pallas_call(
        _outer,
        in_specs=[
            pl.BlockSpec(memory_space=pltpu.MemorySpace.HBM),
            pl.BlockSpec(memory_space=pltpu.MemorySpace.HBM),
        ],
        out_specs=pl.BlockSpec(memory_space=pltpu.MemorySpace.HBM),
        out_shape=jax.ShapeDtypeStruct((T, E), jnp.float32),
    )(x, W)
